# baseline (speedup 1.0000x reference)
import jax
import jax.numpy as jnp
from jax import lax
from jax.experimental import pallas as pl

_N = 320000
_R = 10000
_H = 128
_B = 3200
_NB = _N // _B
_W = 128
_RP = 10240

_F32 = jnp.float32
_I32 = jnp.int32


def _agg_body(seg_ref, trow_ref, tcol_ref, pos16_ref, feat_ref,
              outF_ref, outS_ref):
    b = pl.program_id(0)

    @pl.when(b == 0)
    def _init():
        outF_ref[...] = jnp.zeros_like(outF_ref)
        outS_ref[...] = jnp.zeros_like(outS_ref)

    seg_row = seg_ref[0]
    trow = trow_ref[0]
    tcol = tcol_ref[...]
    pos16 = pos16_ref[...]
    feat = feat_ref[...]

    ca_row = trow == 1

    li = lax.broadcasted_iota(_I32, (1, 16), 1)
    sel = jnp.where(li < 3, 1,
          jnp.where(li < 6, 4,
          jnp.where(li == 6, 1,
          jnp.where(li == 8, 0,
          jnp.where(li == 9, 2,
          jnp.where(li == 10, 4, -1))))))
    c7 = (li == 7).astype(_F32)
    t16 = jnp.broadcast_to(tcol, (_B, 16))
    A = pos16 * (t16 == sel).astype(_F32) + c7

    s0 = seg_ref[0, 0, 0]
    sL = seg_ref[0, 0, _B - 1]
    c0 = (s0 // 8) * 8
    nch = (sL - c0) // _W + 1

    def chunk(c, carry):
        cstart = c0 + c * _W
        iw = lax.broadcasted_iota(_I32, (_W, _B), 0)
        oh = (seg_row - cstart) == iw
        ohf = oh.astype(_F32)
        ohca = jnp.where(ca_row, ohf, 0.0).astype(jnp.bfloat16)
        outF_ref[pl.ds(cstart, _W), :] += jnp.dot(
            ohca, feat, preferred_element_type=_F32)
        outS_ref[pl.ds(cstart, _W), :] += jnp.dot(
            ohf, A, preferred_element_type=_F32)
        return carry

    lax.fori_loop(0, nch, chunk, 0)

    @pl.when(b == pl.num_programs(0) - 1)
    def _finish():
        cnt = outS_ref[:, 6:7]
        outF_ref[...] = outF_ref[...] / jnp.maximum(cnt, 1.0)


def _epi_body(sT_ref, out_ref):
    def row(i):
        return sT_ref[i:i + 1, :]

    ca_cnt = row(6)
    inv_ca = 1.0 / jnp.maximum(ca_cnt, 1.0)
    cax = row(0) * inv_ca
    cay = row(1) * inv_ca
    caz = row(2) * inv_ca
    inv_cb = 1.0 / jnp.maximum(row(10), 1.0)
    cbx = row(3) * inv_cb
    cby = row(4) * inv_cb
    cbz = row(5) * inv_cb
    no_cb = (jnp.abs(cbx) + jnp.abs(cby) + jnp.abs(cbz)) < 1e-6
    cbx = jnp.where(no_cb, cax, cbx)
    cby = jnp.where(no_cb, cay, cby)
    cbz = jnp.where(no_cb, caz, cbz)

    e1x = cbx - cax
    e1y = cby - cay
    e1z = cbz - caz
    n1 = jnp.sqrt(e1x * e1x + e1y * e1y + e1z * e1z)
    d1 = jnp.maximum(n1, 1e-6)
    ux = e1x / d1
    uy = e1y / d1
    uz = e1z / d1
    n2a = jnp.sqrt(ux * ux + uy * uy)
    use_b = n2a < 1e-6
    e2x = jnp.where(use_b, -uz, uy)
    e2y = jnp.where(use_b, 0.0, -ux)
    e2z = jnp.where(use_b, ux, 0.0)
    n2 = jnp.sqrt(e2x * e2x + e2y * e2y + e2z * e2z)
    d2 = jnp.maximum(n2, 1e-6)
    vx = e2x / d2
    vy = e2y / d2
    vz = e2z / d2
    wx = uy * vz - uz * vy
    wy = uz * vx - ux * vz
    wz = ux * vy - uy * vx

    ridx = lax.broadcasted_iota(_I32, ca_cnt.shape, 1)
    valid = (n1 > 1e-6) & (n2 > 1e-6) & (ridx < _R - 1)

    one = jnp.ones_like(cax)
    zero = jnp.zeros_like(cax)
    built = (ux, vx, wx, uy, vy, wy, uz, vz, wz)
    eye = (one, zero, zero, zero, one, zero, zero, zero, one)
    for k in range(9):
        out_ref[k:k + 1, :] = jnp.where(valid, built[k], eye[k])

    mask = (row(7) >= 3.0) & (row(8) > 0.0) & (ca_cnt > 0.0) & (row(9) > 0.0)
    out_ref[9:10, :] = mask.astype(_F32)
    out_ref[10:11, :] = cax
    out_ref[11:12, :] = cay
    out_ref[12:13, :] = caz
    out_ref[13:14, :] = cbx
    out_ref[14:15, :] = cby
    out_ref[15:16, :] = cbz


def kernel(node_features, node_positions, atom_type_ids, segment_ids):
    seg = segment_ids.astype(_I32)
    typ = atom_type_ids.astype(_I32)
    seg3d = seg.reshape(_NB, 1, _B)
    typ3d = typ.reshape(_NB, 1, _B)
    typ_col = typ.reshape(_N, 1)
    ones10 = jnp.ones((_N, 10), dtype=_F32)
    pos16 = jnp.concatenate([node_positions, node_positions, ones10], axis=1)

    outF, outS = pl.pallas_call(
        _agg_body,
        grid=(_NB,),
        in_specs=[
            pl.BlockSpec((1, 1, _B), lambda b: (b, 0, 0)),
            pl.BlockSpec((1, 1, _B), lambda b: (b, 0, 0)),
            pl.BlockSpec((_B, 1), lambda b: (b, 0)),
            pl.BlockSpec((_B, 16), lambda b: (b, 0)),
            pl.BlockSpec((_B, _H), lambda b: (b, 0)),
        ],
        out_specs=[
            pl.BlockSpec((_RP, _H), lambda b: (0, 0)),
            pl.BlockSpec((_RP, 16), lambda b: (0, 0)),
        ],
        out_shape=[
            jax.ShapeDtypeStruct((_RP, _H), _F32),
            jax.ShapeDtypeStruct((_RP, 16), _F32),
        ],
    )(seg3d, typ3d, typ_col, pos16, node_features.astype(jnp.bfloat16))

    sT = outS.T
    outT = pl.pallas_call(
        _epi_body,
        out_shape=jax.ShapeDtypeStruct((16, _RP), _F32),
    )(sT)

    residue_features = outF[:_R]
    pos_CA = outT[10:13, :_R].T
    pos_CB = outT[13:16, :_R].T
    frames = outT[0:9, :_R].T.reshape(_R, 3, 3)
    residue_mask = outT[9, :_R] > 0.5
    return (residue_features, pos_CA, pos_CB, frames, segment_ids,
            residue_mask)
